# baseline (speedup 1.0000x reference)
import functools

import jax
import jax.numpy as jnp
from jax import lax
from jax.experimental import pallas as pl
from jax.experimental.pallas import tpu as pltpu
from jax.experimental.pallas import tpu_sc as plsc

_EPS = 1e-06


def _sc_mesh():
    return plsc.VectorSubcoreMesh(core_axis_name="c", subcore_axis_name="s")


def _worker_id():
    info = plsc.get_sparse_core_info()
    return lax.axis_index("s") * info.num_cores + lax.axis_index("c")


def _sc_gather(table, ids):
    info = plsc.get_sparse_core_info()
    nw = info.num_cores * info.num_subcores
    t, d = ids.shape[0], table.shape[1]
    b_per_w = t // nw

    @functools.partial(
        pl.kernel,
        mesh=_sc_mesh(),
        out_type=jax.ShapeDtypeStruct((t, d), table.dtype),
        scratch_types=[
            pltpu.VMEM((b_per_w,), jnp.int32),
            pltpu.VMEM((b_per_w, d), table.dtype),
            pltpu.SemaphoreType.DMA,
        ],
    )
    def gather_k(table_hbm, idx_hbm, out_hbm, idx_v, rows_v, sem):
        base = _worker_id() * b_per_w
        pltpu.sync_copy(idx_hbm.at[pl.ds(base, b_per_w)], idx_v)
        pltpu.async_copy(table_hbm.at[idx_v], rows_v, sem).wait()
        pltpu.sync_copy(rows_v, out_hbm.at[pl.ds(base, b_per_w)])

    return gather_k(table, ids)


def _route_body(h_ref, g_ref, wr_ref,
                xn_ref, i1_ref, i2_ref, w1_ref, w2_ref):
    h = h_ref[...]
    ms = jnp.mean(h * h, axis=-1, keepdims=True)
    xn = h * lax.rsqrt(ms + _EPS) * (1.0 + g_ref[...])
    logits = jnp.dot(xn, wr_ref[...], preferred_element_type=jnp.float32)
    eidx = lax.broadcasted_iota(jnp.int32, logits.shape, 1)
    i1 = jnp.argmax(logits, axis=-1)[:, None].astype(jnp.int32)
    m1 = jnp.max(logits, axis=-1, keepdims=True)
    masked = jnp.where(eidx == i1, -jnp.inf, logits)
    i2 = jnp.argmax(masked, axis=-1)[:, None].astype(jnp.int32)
    m2 = jnp.max(masked, axis=-1, keepdims=True)
    bb = jnp.exp(m2 - m1)
    w1 = 1.0 / (1.0 + bb)
    i1_ref[...] = i1
    i2_ref[...] = i2
    w1_ref[...] = w1
    w2_ref[...] = 1.0 - w1
    xn_ref[...] = xn.astype(jnp.bfloat16)


def _route(h, gamma, w_router, *, interpret=False):
    t, d = h.shape
    e_num = w_router.shape[1]
    full = lambda *s: pl.BlockSpec(s, lambda: tuple(0 for _ in s))
    return pl.pallas_call(
        _route_body,
        in_specs=[full(t, d), full(1, d), full(d, e_num)],
        out_specs=[full(t, d), full(t, 1), full(t, 1), full(t, 1), full(t, 1)],
        out_shape=[
            jax.ShapeDtypeStruct((t, d), jnp.bfloat16),
            jax.ShapeDtypeStruct((t, 1), jnp.int32),
            jax.ShapeDtypeStruct((t, 1), jnp.int32),
            jax.ShapeDtypeStruct((t, 1), jnp.float32),
            jax.ShapeDtypeStruct((t, 1), jnp.float32),
        ],
        interpret=interpret,
    )(h, gamma, w_router)


def _moe_body(xn_ref, h_ref, i1_ref, i2_ref, w1_ref, w2_ref,
              wg_ref, wu_ref, wd_ref, out_ref, acc_ref, *, tb, e_num):
    e = pl.program_id(0)
    i = pl.program_id(1)
    sl = pl.ds(i * tb, tb)
    x = xn_ref[sl, :]
    g = jnp.dot(x, wg_ref[0], preferred_element_type=jnp.float32)
    u = jnp.dot(x, wu_ref[0], preferred_element_type=jnp.float32)
    ge = g * jax.nn.sigmoid(g) * u
    wsel = (w1_ref[sl, :] * (i1_ref[sl, :] == e).astype(jnp.float32)
            + w2_ref[sl, :] * (i2_ref[sl, :] == e).astype(jnp.float32))
    yw = (ge * wsel).astype(jnp.bfloat16)
    contrib = jnp.dot(yw, wd_ref[0], preferred_element_type=jnp.float32)

    @pl.when(e == 0)
    def _init():
        acc_ref[sl, :] = h_ref[sl, :] + contrib

    @pl.when(e > 0)
    def _acc():
        acc_ref[sl, :] += contrib

    @pl.when(e == e_num - 1)
    def _emit():
        out_ref[sl, :] = acc_ref[sl, :]


def _moe(xn, h, i1, i2, w1, w2, wg, wu, wd, *, interpret=False, nt=1):
    t, d = h.shape
    e_num, _, f = wg.shape
    tb = t // nt
    full = lambda *s: pl.BlockSpec(s, lambda e, i: tuple(0 for _ in s))
    return pl.pallas_call(
        functools.partial(_moe_body, tb=tb, e_num=e_num),
        grid=(e_num, nt),
        in_specs=[
            full(t, d),
            full(t, d),
            full(t, 1),
            full(t, 1),
            full(t, 1),
            full(t, 1),
            pl.BlockSpec((1, d, f), lambda e, i: (e, 0, 0)),
            pl.BlockSpec((1, d, f), lambda e, i: (e, 0, 0)),
            pl.BlockSpec((1, f, d), lambda e, i: (e, 0, 0)),
        ],
        out_specs=full(t, d),
        out_shape=jax.ShapeDtypeStruct((t, d), jnp.float32),
        scratch_shapes=[pltpu.VMEM((t, d), jnp.float32)],
        interpret=interpret,
    )(xn, h, i1, i2, w1, w2, wg, wu, wd)


def kernel(input_ids, embed_table, norm_gamma, w_router, w_gate, w_up, w_down):
    b, s = input_ids.shape
    d = embed_table.shape[1]
    ids = input_ids.reshape(-1).astype(jnp.int32)
    h = _sc_gather(embed_table, ids)
    xn, i1, i2, w1, w2 = _route(h, norm_gamma.reshape(1, d), w_router)
    out = _moe(xn, h, i1, i2, w1, w2,
               w_gate.astype(jnp.bfloat16), w_up.astype(jnp.bfloat16),
               w_down.astype(jnp.bfloat16))
    return out.reshape(b, s, d)


# speedup vs baseline: 1.0849x; 1.0849x over previous
import functools

import jax
import jax.numpy as jnp
from jax import lax
from jax.experimental import pallas as pl
from jax.experimental.pallas import tpu as pltpu
from jax.experimental.pallas import tpu_sc as plsc

_EPS = 1e-06


def _sc_gather(table, ids):
    info = plsc.get_sparse_core_info()
    nw = info.num_cores * info.num_subcores
    t, d = ids.shape[0], table.shape[1]
    b_per_w = t // nw
    mesh = plsc.VectorSubcoreMesh(core_axis_name="c", subcore_axis_name="s")

    @functools.partial(
        pl.kernel,
        mesh=mesh,
        out_type=jax.ShapeDtypeStruct((t, d), table.dtype),
        scratch_types=[
            pltpu.VMEM((b_per_w,), jnp.int32),
            pltpu.VMEM((b_per_w, d), table.dtype),
            pltpu.SemaphoreType.DMA,
        ],
    )
    def gather_k(table_hbm, idx_hbm, out_hbm, idx_v, rows_v, sem):
        wid = lax.axis_index("s") * info.num_cores + lax.axis_index("c")
        base = wid * b_per_w
        pltpu.sync_copy(idx_hbm.at[pl.ds(base, b_per_w)], idx_v)
        pltpu.async_copy(table_hbm.at[idx_v], rows_v, sem).wait()
        pltpu.sync_copy(rows_v, out_hbm.at[pl.ds(base, b_per_w)])

    return gather_k(table, ids)


def _moe_body(h_ref, g_ref, wr_ref, wg_ref, wu_ref, wd_ref, out_ref,
              xn_ref, cmb_ref):
    e = pl.program_id(1)

    @pl.when(e == 0)
    def _prologue():
        h = h_ref[...]
        ms = jnp.mean(h * h, axis=-1, keepdims=True)
        xn = h * lax.rsqrt(ms + _EPS) * (1.0 + g_ref[...])
        logits = jnp.dot(xn, wr_ref[...], preferred_element_type=jnp.float32)
        eidx = lax.broadcasted_iota(jnp.int32, logits.shape, 1)
        i1 = jnp.argmax(logits, axis=-1)[:, None].astype(jnp.int32)
        m1 = jnp.max(logits, axis=-1, keepdims=True)
        masked = jnp.where(eidx == i1, -jnp.inf, logits)
        i2 = jnp.argmax(masked, axis=-1)[:, None].astype(jnp.int32)
        m2 = jnp.max(masked, axis=-1, keepdims=True)
        bb = jnp.exp(m2 - m1)
        w1 = 1.0 / (1.0 + bb)
        cmb_ref[...] = jnp.concatenate(
            [w1, 1.0 - w1, i1.astype(jnp.float32), i2.astype(jnp.float32)],
            axis=1)
        xn_ref[...] = xn.astype(jnp.bfloat16)
        out_ref[...] = h

    xn = xn_ref[...]
    cmb = cmb_ref[...]
    ef = e.astype(jnp.float32)
    wsel = (cmb[:, 0:1] * (cmb[:, 2:3] == ef).astype(jnp.float32)
            + cmb[:, 1:2] * (cmb[:, 3:4] == ef).astype(jnp.float32))
    g = jnp.dot(xn, wg_ref[0], preferred_element_type=jnp.float32)
    u = jnp.dot(xn, wu_ref[0], preferred_element_type=jnp.float32)
    ge = g * jax.nn.sigmoid(g) * u
    gw = (ge * wsel).astype(jnp.bfloat16)
    out_ref[...] += jnp.dot(gw, wd_ref[0], preferred_element_type=jnp.float32)


def _moe(h, gamma, w_router, wg, wu, wd, *, interpret=False, tb=2048):
    t, d = h.shape
    e_num, _, f = wg.shape
    return pl.pallas_call(
        _moe_body,
        grid=(t // tb, e_num),
        in_specs=[
            pl.BlockSpec((tb, d), lambda i, e: (i, 0)),
            pl.BlockSpec((1, d), lambda i, e: (0, 0)),
            pl.BlockSpec((d, e_num), lambda i, e: (0, 0)),
            pl.BlockSpec((1, d, f), lambda i, e: (e, 0, 0)),
            pl.BlockSpec((1, d, f), lambda i, e: (e, 0, 0)),
            pl.BlockSpec((1, f, d), lambda i, e: (e, 0, 0)),
        ],
        out_specs=pl.BlockSpec((tb, d), lambda i, e: (i, 0)),
        out_shape=jax.ShapeDtypeStruct((t, d), jnp.float32),
        scratch_shapes=[
            pltpu.VMEM((tb, d), jnp.bfloat16),
            pltpu.VMEM((tb, 4), jnp.float32),
        ],
        interpret=interpret,
    )(h, gamma, w_router, wg, wu, wd)


def kernel(input_ids, embed_table, norm_gamma, w_router, w_gate, w_up, w_down):
    b, s = input_ids.shape
    d = embed_table.shape[1]
    ids = input_ids.reshape(-1).astype(jnp.int32)
    h = _sc_gather(embed_table, ids)
    out = _moe(h, norm_gamma.reshape(1, d), w_router,
               w_gate.astype(jnp.bfloat16), w_up.astype(jnp.bfloat16),
               w_down.astype(jnp.bfloat16))
    return out.reshape(b, s, d)


# speedup vs baseline: 1.1030x; 1.0166x over previous
import functools

import jax
import jax.numpy as jnp
from jax import lax
from jax.experimental import pallas as pl
from jax.experimental.pallas import tpu as pltpu
from jax.experimental.pallas import tpu_sc as plsc

_EPS = 1e-06


def _sc_gather(table, ids):
    info = plsc.get_sparse_core_info()
    nw = info.num_cores * info.num_subcores
    t, d = ids.shape[0], table.shape[1]
    b_per_w = t // nw
    mesh = plsc.VectorSubcoreMesh(core_axis_name="c", subcore_axis_name="s")

    @functools.partial(
        pl.kernel,
        mesh=mesh,
        out_type=jax.ShapeDtypeStruct((t, d), table.dtype),
        scratch_types=[
            pltpu.VMEM((b_per_w,), jnp.int32),
            pltpu.VMEM((b_per_w, d), table.dtype),
            pltpu.SemaphoreType.DMA,
        ],
    )
    def gather_k(table_hbm, idx_hbm, out_hbm, idx_v, rows_v, sem):
        wid = lax.axis_index("s") * info.num_cores + lax.axis_index("c")
        base = wid * b_per_w
        pltpu.sync_copy(idx_hbm.at[pl.ds(base, b_per_w)], idx_v)
        pltpu.async_copy(table_hbm.at[idx_v], rows_v, sem).wait()
        pltpu.sync_copy(rows_v, out_hbm.at[pl.ds(base, b_per_w)])

    return gather_k(table, ids)


def _moe_body(h_ref, g_ref, wr_ref, wg_ref, wu_ref, wd_ref, out_ref,
              xn_ref, cmb_ref):
    e = pl.program_id(1)

    @pl.when(e == 0)
    def _prologue():
        h = h_ref[...]
        ms = jnp.mean(h * h, axis=-1, keepdims=True)
        xn = h * lax.rsqrt(ms + _EPS) * (1.0 + g_ref[...])
        logits = jnp.dot(xn, wr_ref[...], preferred_element_type=jnp.float32)
        eidx = lax.broadcasted_iota(jnp.int32, logits.shape, 1)
        i1 = jnp.argmax(logits, axis=-1)[:, None].astype(jnp.int32)
        m1 = jnp.max(logits, axis=-1, keepdims=True)
        masked = jnp.where(eidx == i1, -jnp.inf, logits)
        i2 = jnp.argmax(masked, axis=-1)[:, None].astype(jnp.int32)
        m2 = jnp.max(masked, axis=-1, keepdims=True)
        bb = jnp.exp(m2 - m1)
        w1 = 1.0 / (1.0 + bb)
        cmb_ref[...] = jnp.concatenate(
            [w1, 1.0 - w1, i1.astype(jnp.float32), i2.astype(jnp.float32)],
            axis=1)
        xn_ref[...] = xn.astype(jnp.bfloat16)
        out_ref[...] = h

    xn = xn_ref[...]
    cmb = cmb_ref[...]
    ef = e.astype(jnp.float32)
    wsel = (cmb[:, 0:1] * (cmb[:, 2:3] == ef).astype(jnp.float32)
            + cmb[:, 1:2] * (cmb[:, 3:4] == ef).astype(jnp.float32))
    g = jnp.dot(xn, wg_ref[0], preferred_element_type=jnp.float32)
    u = jnp.dot(xn, wu_ref[0], preferred_element_type=jnp.float32)
    ge = g * jax.nn.sigmoid(g) * u
    gw = (ge * wsel).astype(jnp.bfloat16)
    out_ref[...] += jnp.dot(gw, wd_ref[0], preferred_element_type=jnp.float32)


def _moe(h, gamma, w_router, wg, wu, wd, *, interpret=False, tb=1024):
    t, d = h.shape
    e_num, _, f = wg.shape
    return pl.pallas_call(
        _moe_body,
        grid=(t // tb, e_num),
        in_specs=[
            pl.BlockSpec((tb, d), lambda i, e: (i, 0)),
            pl.BlockSpec((1, d), lambda i, e: (0, 0)),
            pl.BlockSpec((d, e_num), lambda i, e: (0, 0)),
            pl.BlockSpec((1, d, f), lambda i, e: (e, 0, 0)),
            pl.BlockSpec((1, d, f), lambda i, e: (e, 0, 0)),
            pl.BlockSpec((1, f, d), lambda i, e: (e, 0, 0)),
        ],
        out_specs=pl.BlockSpec((tb, d), lambda i, e: (i, 0)),
        out_shape=jax.ShapeDtypeStruct((t, d), jnp.float32),
        scratch_shapes=[
            pltpu.VMEM((tb, d), jnp.bfloat16),
            pltpu.VMEM((tb, 4), jnp.float32),
        ],
        interpret=interpret,
    )(h, gamma, w_router, wg, wu, wd)


def kernel(input_ids, embed_table, norm_gamma, w_router, w_gate, w_up, w_down):
    b, s = input_ids.shape
    d = embed_table.shape[1]
    ids = input_ids.reshape(-1).astype(jnp.int32)
    h = _sc_gather(embed_table, ids)
    out = _moe(h, norm_gamma.reshape(1, d), w_router,
               w_gate.astype(jnp.bfloat16), w_up.astype(jnp.bfloat16),
               w_down.astype(jnp.bfloat16))
    return out.reshape(b, s, d)
